# Initial kernel scaffold; baseline (speedup 1.0000x reference)
import jax
import jax.numpy as jnp
from jax.experimental import pallas as pl


def kernel(x, edge_index, edge_attr, batch, node_emb, edge_emb, gine_lin_W, gine_lin_b, gine_W1, gine_b1, gine_W2, gine_b2, bn_g, bn_b, vn_W1, vn_b1, vn_g1, vn_be1, vn_W2, vn_b2, vn_g2, vn_be2, head_W1, head_b1, head_W2, head_b2, head_W3, head_b3):
    raise NotImplementedError("write your pallas kernel here")


# speedup vs baseline: 2.7166x; 2.7166x over previous
import functools

import jax
import jax.numpy as jnp
from jax import lax
from jax.experimental import pallas as pl
from jax.experimental.pallas import tpu as pltpu
from jax.experimental.pallas import tpu_sc as plsc

_N = 10000
_E = 160000
_D = 256
_H = 128
_G = 64
_L = 4
_OUT = 11

_NC = 2
_NS = 16
_EPT = _E // _NS
_GROUP = 128
_TGT = 10240
_NGRP = _TGT // _GROUP
_APAD = 10240
_ROWS_PT = _APAD // _NS

_R = 2000
_NB = _N // _R


def _edge_body(z_hbm, gidx_hbm, didx_hbm, zero_hbm, out_hbm,
               gidx_v, didx_v, rows_v, aggr_sh, sem):
    c = lax.axis_index("c")
    s = lax.axis_index("s")
    pltpu.sync_copy(zero_hbm.at[pl.ds(s * _ROWS_PT, _ROWS_PT)],
                    aggr_sh.at[pl.ds(s * _ROWS_PT, _ROWS_PT)])
    pltpu.sync_copy(gidx_hbm.at[pl.ds((c * _NS + s) * _NGRP, _NGRP)], gidx_v)
    pltpu.sync_copy(didx_hbm.at[pl.ds(s * _NGRP, _NGRP)], didx_v)
    plsc.subcore_barrier()
    for g in range(_NGRP):
        pltpu.async_copy(z_hbm.at[gidx_v.at[g]], rows_v, sem).wait()
        pltpu.sync_copy(rows_v, aggr_sh.at[didx_v.at[g]], add=True)
    plsc.subcore_barrier()
    pltpu.sync_copy(aggr_sh.at[pl.ds(s * _ROWS_PT, _ROWS_PT)],
                    out_hbm.at[pl.ds(c * _APAD + s * _ROWS_PT, _ROWS_PT)])


@functools.cache
def _edge_kernel():
    return functools.partial(
        pl.kernel,
        out_type=jax.ShapeDtypeStruct((2 * _APAD, _H), jnp.float32),
        mesh=plsc.VectorSubcoreMesh(core_axis_name="c", subcore_axis_name="s",
                                    num_cores=_NC, num_subcores=_NS),
        scratch_types=[
            pltpu.VMEM((_NGRP, _GROUP), jnp.int32),
            pltpu.VMEM((_NGRP, _GROUP), jnp.int32),
            pltpu.VMEM((_GROUP, _H), jnp.float32),
            pltpu.VMEM_SHARED((_APAD, _H), jnp.float32),
            pltpu.SemaphoreType.DMA,
        ],
    )(_edge_body)


def _edge_aggregate(z2d, gidx, didx, zeros_h):
    return _edge_kernel()(z2d, gidx, didx, zeros_h)


def _write_z(z_ref, hv, tbl):
    for a in range(4):
        za = jnp.maximum(hv + tbl[a:a + 1, :], 0.0)
        for c in range(2):
            z_ref[c * 4 + a] = za[:, c * _H:(c + 1) * _H]


def _prep0_body(x_ref, nemb_ref, eemb_ref, lw_ref, lb_ref, hv_ref, z_ref):
    onehot = (x_ref[:, :] == lax.broadcasted_iota(jnp.int32, (_R, 32), 1)
              ).astype(jnp.float32)
    h = jnp.dot(onehot, nemb_ref[:, :], preferred_element_type=jnp.float32,
                precision=lax.Precision.HIGHEST)
    tbl = jnp.dot(eemb_ref[:, :], lw_ref[:, :],
                  preferred_element_type=jnp.float32) + lb_ref[:, :]
    hv_ref[:, :] = h
    _write_z(z_ref, h, tbl)


def _prep_body(h_ref, b_ref, vn_ref, eemb_ref, lw_ref, lb_ref, hv_ref, z_ref):
    onehot = (b_ref[:, :] == lax.broadcasted_iota(jnp.int32, (_R, _G), 1)
              ).astype(jnp.float32)
    hv = h_ref[:, :] + jnp.dot(onehot, vn_ref[:, :],
                               preferred_element_type=jnp.float32,
                               precision=lax.Precision.HIGHEST)
    tbl = jnp.dot(eemb_ref[:, :], lw_ref[:, :],
                  preferred_element_type=jnp.float32) + lb_ref[:, :]
    hv_ref[:, :] = hv
    _write_z(z_ref, hv, tbl)


def _mlp_body(hv_ref, ag0_ref, ag1_ref, w1_ref, b1_ref, w2_ref, b2_ref,
              y_ref, ps_ref, pq_ref):
    out = hv_ref[:, :] + jnp.concatenate([ag0_ref[0], ag1_ref[0]], axis=1)
    y = jnp.maximum(jnp.dot(out, w1_ref[:, :],
                            preferred_element_type=jnp.float32) + b1_ref[:, :],
                    0.0)
    y = jnp.dot(y, w2_ref[:, :], preferred_element_type=jnp.float32) \
        + b2_ref[:, :]
    y_ref[:, :] = y
    ps_ref[0] = jnp.sum(y, axis=0, keepdims=True)
    pq_ref[0] = jnp.sum(y * y, axis=0, keepdims=True)


def _bnleaky_body(y_ref, ps_ref, pq_ref, g_ref, bb_ref, bt_ref,
                  hn_ref, pool_ref):

    s = ps_ref[0]
    q = pq_ref[0]
    for i in range(1, _NB):
        s = s + ps_ref[i]
        q = q + pq_ref[i]
    mu = s * (1.0 / _N)
    var = q * (1.0 / _N) - mu * mu
    y = y_ref[:, :]
    yb = g_ref[:, :] * (y - mu) / jnp.sqrt(var + 1e-5) + bb_ref[:, :]
    hn = jnp.where(yb >= 0.0, yb, 0.1 * yb)
    hn_ref[:, :] = hn
    onehot_t = (lax.broadcasted_iota(jnp.int32, (_G, _R), 0) == bt_ref[0]
                ).astype(jnp.float32)
    pool_ref[0] = jnp.dot(onehot_t, hn, preferred_element_type=jnp.float32,
                          precision=lax.Precision.HIGHEST)


def _bn64(y, g, b, eps=1e-5):
    mu = jnp.mean(y, axis=0, keepdims=True)
    var = jnp.mean((y - mu) * (y - mu), axis=0, keepdims=True)
    return g * (y - mu) / jnp.sqrt(var + eps) + b


def _vnmlp_body(pool_ref, vn_ref, vw1_ref, vb1_ref, vg1_ref, vbe1_ref,
                vw2_ref, vb2_ref, vg2_ref, vbe2_ref, vn_out):
    pooled = vn_ref[:, :]
    for i in range(_NB):
        pooled = pooled + pool_ref[i]
    t = jnp.dot(pooled, vw1_ref[:, :], preferred_element_type=jnp.float32) \
        + vb1_ref[:, :]
    t = jnp.maximum(_bn64(t, vg1_ref[:, :], vbe1_ref[:, :]), 0.0)
    t = jnp.dot(t, vw2_ref[:, :], preferred_element_type=jnp.float32) \
        + vb2_ref[:, :]
    vn_out[:, :] = jnp.maximum(_bn64(t, vg2_ref[:, :], vbe2_ref[:, :]), 0.0)


def _head_body(pool_ref, bt_ref, hw1_ref, hb1_ref, hw2_ref, hb2_ref,
               hw3_ref, hb3_ref, o_ref):
    pooled = pool_ref[0]
    for i in range(1, _NB):
        pooled = pooled + pool_ref[i]
    onehot_t = (lax.broadcasted_iota(jnp.int32, (_G, _N), 0) == bt_ref[:, :]
                ).astype(jnp.float32)
    counts = jnp.dot(onehot_t, jnp.ones((_N, 1), jnp.float32),
                     preferred_element_type=jnp.float32,
                     precision=lax.Precision.HIGHEST)
    xg = pooled / jnp.maximum(counts, 1.0)
    o = jnp.maximum(jnp.dot(xg, hw1_ref[:, :],
                            preferred_element_type=jnp.float32) + hb1_ref[:, :],
                    0.0)
    o = jnp.maximum(jnp.dot(o, hw2_ref[:, :],
                            preferred_element_type=jnp.float32) + hb2_ref[:, :],
                    0.0)
    o_ref[:, :] = jnp.dot(o, hw3_ref[:, :],
                          preferred_element_type=jnp.float32) + hb3_ref[:, :]


def _full(shape):
    ix = tuple(0 for _ in shape)
    return pl.BlockSpec(shape, lambda i, _ix=ix: _ix)


_f32 = jnp.float32

_prep0 = pl.pallas_call(
    _prep0_body,
    grid=(_NB,),
    in_specs=[pl.BlockSpec((_R, 1), lambda i: (i, 0)),
              _full((32, _D)), _full((4, _D)), _full((_D, _D)),
              _full((1, _D))],
    out_specs=[pl.BlockSpec((_R, _D), lambda i: (i, 0)),
               pl.BlockSpec((8, _R, _H), lambda i: (0, i, 0))],
    out_shape=(jax.ShapeDtypeStruct((_N, _D), _f32),
               jax.ShapeDtypeStruct((8, _N, _H), _f32)),
)

_prep = pl.pallas_call(
    _prep_body,
    grid=(_NB,),
    in_specs=[pl.BlockSpec((_R, _D), lambda i: (i, 0)),
              pl.BlockSpec((_R, 1), lambda i: (i, 0)),
              _full((_G, _D)), _full((4, _D)), _full((_D, _D)),
              _full((1, _D))],
    out_specs=[pl.BlockSpec((_R, _D), lambda i: (i, 0)),
               pl.BlockSpec((8, _R, _H), lambda i: (0, i, 0))],
    out_shape=(jax.ShapeDtypeStruct((_N, _D), _f32),
               jax.ShapeDtypeStruct((8, _N, _H), _f32)),
)

_mlp = pl.pallas_call(
    _mlp_body,
    grid=(_NB,),
    in_specs=[pl.BlockSpec((_R, _D), lambda i: (i, 0)),
              pl.BlockSpec((1, _R, _H), lambda i: (0, i, 0)),
              pl.BlockSpec((1, _R, _H), lambda i: (1, i, 0)),
              _full((_D, _D)), _full((1, _D)), _full((_D, _D)),
              _full((1, _D))],
    out_specs=[pl.BlockSpec((_R, _D), lambda i: (i, 0)),
               pl.BlockSpec((1, 1, _D), lambda i: (i, 0, 0)),
               pl.BlockSpec((1, 1, _D), lambda i: (i, 0, 0))],
    out_shape=(jax.ShapeDtypeStruct((_N, _D), _f32),
               jax.ShapeDtypeStruct((_NB, 1, _D), _f32),
               jax.ShapeDtypeStruct((_NB, 1, _D), _f32)),
)

_bnleaky = pl.pallas_call(
    _bnleaky_body,
    grid=(_NB,),
    in_specs=[pl.BlockSpec((_R, _D), lambda i: (i, 0)),
              _full((_NB, 1, _D)), _full((_NB, 1, _D)),
              _full((1, _D)), _full((1, _D)),
              pl.BlockSpec((1, 1, _R), lambda i: (i, 0, 0))],
    out_specs=[pl.BlockSpec((_R, _D), lambda i: (i, 0)),
               pl.BlockSpec((1, _G, _D), lambda i: (i, 0, 0))],
    out_shape=(jax.ShapeDtypeStruct((_N, _D), _f32),
               jax.ShapeDtypeStruct((_NB, _G, _D), _f32)),
)

_vnmlp = pl.pallas_call(
    _vnmlp_body,
    out_shape=jax.ShapeDtypeStruct((_G, _D), _f32),
)

_head = pl.pallas_call(
    _head_body,
    out_shape=jax.ShapeDtypeStruct((_G, _OUT), _f32),
)


def kernel(x, edge_index, edge_attr, batch, node_emb, edge_emb, gine_lin_W,
           gine_lin_b, gine_W1, gine_b1, gine_W2, gine_b2, bn_g, bn_b,
           vn_W1, vn_b1, vn_g1, vn_be1, vn_W2, vn_b2, vn_g2, vn_be2,
           head_W1, head_b1, head_W2, head_b2, head_W3, head_b3):
    i32 = jnp.int32
    x2 = x.reshape(_N, 1).astype(i32)
    batch_col = batch.reshape(_N, 1).astype(i32)
    batch_row = batch.reshape(1, _N).astype(i32)
    batch_blk = batch.reshape(_NB, 1, _R).astype(i32)
    src = edge_index[0].astype(i32)
    dst = edge_index[1].astype(i32)
    attr = edge_attr.astype(i32)
    base_t = (attr * _N + src).reshape(_NS, _EPT)
    dst_t = dst.reshape(_NS, _EPT)
    padc = _TGT - _EPT
    g0 = jnp.pad(base_t, ((0, 0), (0, padc)))
    g1 = jnp.pad(base_t + 4 * _N, ((0, 0), (0, padc)))
    gidx = jnp.concatenate([g0, g1]).reshape(2 * _NS * _NGRP, _GROUP)
    didx = jnp.pad(dst_t, ((0, 0), (0, padc)),
                   constant_values=_N).reshape(_NS * _NGRP, _GROUP)
    zeros_h = jnp.zeros((_APAD, _H), _f32)
    nemb_p = jnp.zeros((32, _D), _f32).at[:28, :].set(node_emb)
    r = lambda v: v.reshape(1, -1)

    hv, z = _prep0(x2, nemb_p, edge_emb, gine_lin_W[0], r(gine_lin_b[0]))
    vn = jnp.zeros((_G, _D), _f32)
    for l in range(_L):
        aggr = _edge_aggregate(z.reshape(8 * _N, _H), gidx, didx, zeros_h)
        ag3 = aggr.reshape(2, _APAD, _H)
        y, ps, pq = _mlp(hv, ag3, ag3, gine_W1[l], r(gine_b1[l]),
                         gine_W2[l], r(gine_b2[l]))
        hn, pool = _bnleaky(y, ps, pq, r(bn_g[l]), r(bn_b[l]), batch_blk)
        if l < _L - 1:
            vn = _vnmlp(pool, vn, vn_W1[l], r(vn_b1[l]), r(vn_g1[l]),
                        r(vn_be1[l]), vn_W2[l], r(vn_b2[l]), r(vn_g2[l]),
                        r(vn_be2[l]))
            hv, z = _prep(hn, batch_col, vn, edge_emb, gine_lin_W[l + 1],
                          r(gine_lin_b[l + 1]))
        else:
            o = _head(pool, batch_row, head_W1, r(head_b1), head_W2,
                      r(head_b2), head_W3, r(head_b3))
    return o
